# baseline (speedup 1.0000x reference)
import functools

import jax
import jax.numpy as jnp
from jax import lax
from jax.experimental import pallas as pl
from jax.experimental.pallas import tpu as pltpu
from jax.experimental.pallas import tpu_sc as plsc

_INFO = plsc.get_sparse_core_info()
_NC = _INFO.num_cores
_NS = _INFO.num_subcores
_NW = _NC * _NS
_L = _INFO.num_lanes

_D = 128
_CHUNK = 512
_IDX_ROWS = _CHUNK // 128


def _sc_gather(table_hbm, idx_hbm, out_hbm, idx_v, rows_v, sem):
    wid = lax.axis_index("s") * _NC + lax.axis_index("c")
    n_idx_rows = idx_hbm.shape[0]
    rows_per_w = n_idx_rows // _NW
    n_chunks = rows_per_w // _IDX_ROWS

    def body(chunk_i, _):
        idx_row_base = wid * rows_per_w + chunk_i * _IDX_ROWS
        pltpu.sync_copy(idx_hbm.at[pl.ds(idx_row_base, _IDX_ROWS)], idx_v)
        copies = []
        for j in range(_IDX_ROWS):
            copies.append(pltpu.async_copy(
                table_hbm.at[idx_v.at[j]],
                rows_v.at[pl.ds(j * 128, 128)],
                sem))
        for c in copies:
            c.wait()
        out_base = idx_row_base * 128
        pltpu.sync_copy(rows_v, out_hbm.at[pl.ds(out_base, _CHUNK)])
        return ()

    lax.fori_loop(0, n_chunks, body, ())


def kernel(y, w1):
    B = y.shape[0] * y.shape[1]
    y_flat = y.reshape(B // 128, 128)

    mesh = plsc.VectorSubcoreMesh(core_axis_name="c", subcore_axis_name="s")
    run = functools.partial(
        pl.kernel,
        mesh=mesh,
        out_type=jax.ShapeDtypeStruct((B, _D), jnp.float32),
        scratch_types=[
            pltpu.VMEM((_IDX_ROWS, 128), jnp.int32),
            pltpu.VMEM((_CHUNK, _D), jnp.float32),
            pltpu.SemaphoreType.DMA,
        ],
    )(_sc_gather)
    out = run(w1, y_flat)
    return out.reshape(y.shape[0], y.shape[1], _D)


# speedup vs baseline: 5.2129x; 5.2129x over previous
import functools

import jax
import jax.numpy as jnp
from jax import lax
from jax.experimental import pallas as pl
from jax.experimental.pallas import tpu as pltpu
from jax.experimental.pallas import tpu_sc as plsc

_INFO = plsc.get_sparse_core_info()
_NC = _INFO.num_cores
_NS = _INFO.num_subcores
_NW = _NC * _NS
_L = _INFO.num_lanes

_D = 128
_C = 400


def _sc_embed(table_hbm, idx_hbm, out_hbm,
              table_v, ibuf0, ibuf1, st0, st1, si0, si1, so0, so1):
    wid = lax.axis_index("s") * _NC + lax.axis_index("c")
    B = idx_hbm.shape[0]
    per_w = B // _NW
    n = per_w // _C
    base = wid * per_w

    pltpu.sync_copy(table_hbm, table_v)

    bufs = ((ibuf0, st0, si0, so0), (ibuf1, st1, si1, so1))

    def idx_copy(g, ib, sem):
        return pltpu.make_async_copy(
            idx_hbm.at[pl.ds(base + g * _C, _C)], ib, sem)

    def out_copy(g, sb, sem):
        return pltpu.make_async_copy(
            sb, out_hbm.at[pl.ds(base + g * _C, _C)], sem)

    def expand(ib, sb):
        def row16(r16, _):
            r = r16 * _L
            tv = ib[pl.ds(r, _L)]
            for u in range(_L):
                t = tv[u]
                for k in range(_D // _L):
                    sb[r + u, pl.ds(k * _L, _L)] = table_v[t, pl.ds(k * _L, _L)]
            return ()
        lax.fori_loop(0, _C // _L, row16, ())

    idx_copy(0, ibuf0, si0).start()
    idx_copy(1, ibuf1, si1).start()

    for b in range(2):
        ib, sb, si, so = bufs[b]
        idx_copy(b, ib, si).wait()
        expand(ib, sb)
        out_copy(b, sb, so).start()
        idx_copy(b + 2, ib, si).start()

    def pair(p, _):
        for b in range(2):
            g = 2 + 2 * p + b
            ib, sb, si, so = bufs[b]
            idx_copy(g, ib, si).wait()
            out_copy(g - 2, sb, so).wait()
            expand(ib, sb)
            out_copy(g, sb, so).start()
            idx_copy(g + 2, ib, si).start()
        return ()
    lax.fori_loop(0, (n - 4) // 2, pair, ())

    for b in range(2):
        g = n - 2 + b
        ib, sb, si, so = bufs[b]
        idx_copy(g, ib, si).wait()
        out_copy(g - 2, sb, so).wait()
        expand(ib, sb)
        out_copy(g, sb, so).start()
    for b in range(2):
        ib, sb, si, so = bufs[b]
        out_copy(n - 2 + b, sb, so).wait()


def kernel(y, w1):
    B = y.shape[0] * y.shape[1]
    y_flat = y.reshape(B)

    mesh = plsc.VectorSubcoreMesh(core_axis_name="c", subcore_axis_name="s")
    run = functools.partial(
        pl.kernel,
        mesh=mesh,
        out_type=jax.ShapeDtypeStruct((B, _D), jnp.float32),
        scratch_types=[
            pltpu.VMEM((10, _D), jnp.float32),
            pltpu.VMEM((_C,), jnp.int32),
            pltpu.VMEM((_C,), jnp.int32),
            pltpu.VMEM((_C, _D), jnp.float32),
            pltpu.VMEM((_C, _D), jnp.float32),
            pltpu.SemaphoreType.DMA,
            pltpu.SemaphoreType.DMA,
            pltpu.SemaphoreType.DMA,
            pltpu.SemaphoreType.DMA,
        ],
    )(_sc_embed)
    out = run(w1, y_flat)
    return out.reshape(y.shape[0], y.shape[1], _D)


# speedup vs baseline: 18.0234x; 3.4574x over previous
import functools

import jax
import jax.numpy as jnp
from jax import lax
from jax.experimental import pallas as pl
from jax.experimental.pallas import tpu as pltpu
from jax.experimental.pallas import tpu_sc as plsc

_INFO = plsc.get_sparse_core_info()
_NC = _INFO.num_cores
_NS = _INFO.num_subcores
_NW = _NC * _NS
_L = _INFO.num_lanes

_D = 128
_C = 400


def _sc_embed(table_hbm, idx_hbm, out_hbm,
              table_v, ibuf0, ibuf1, st0, st1, si0, si1, so0, so1):
    wid = lax.axis_index("s") * _NC + lax.axis_index("c")
    B = idx_hbm.shape[0]
    per_w = B // _NW
    n = per_w // _C
    base = wid * per_w

    pltpu.sync_copy(table_hbm, table_v)

    bufs = ((ibuf0, st0, si0, so0), (ibuf1, st1, si1, so1))

    def idx_copy(g, ib, sem):
        return pltpu.make_async_copy(
            idx_hbm.at[pl.ds(base + g * _C, _C)], ib, sem)

    def out_copy(g, sb, sem):
        return pltpu.make_async_copy(
            sb, out_hbm.at[pl.ds(base + g * _C, _C)], sem)

    def expand(ib, sb):
        @plsc.parallel_loop(0, _C // _L, unroll=2)
        def row16(r16):
            r = r16 * _L
            tv = ib[pl.ds(r, _L)]
            for u in range(_L):
                t = tv[u]
                vals = [table_v[t, pl.ds(k * _L, _L)] for k in range(_D // _L)]
                for k in range(_D // _L):
                    sb[r + u, pl.ds(k * _L, _L)] = vals[k]

    idx_copy(0, ibuf0, si0).start()
    idx_copy(1, ibuf1, si1).start()

    for b in range(2):
        ib, sb, si, so = bufs[b]
        idx_copy(b, ib, si).wait()
        expand(ib, sb)
        out_copy(b, sb, so).start()
        idx_copy(b + 2, ib, si).start()

    def pair(p, _):
        for b in range(2):
            g = 2 + 2 * p + b
            ib, sb, si, so = bufs[b]
            idx_copy(g, ib, si).wait()
            out_copy(g - 2, sb, so).wait()
            expand(ib, sb)
            out_copy(g, sb, so).start()
            idx_copy(g + 2, ib, si).start()
        return ()
    lax.fori_loop(0, (n - 4) // 2, pair, ())

    for b in range(2):
        g = n - 2 + b
        ib, sb, si, so = bufs[b]
        idx_copy(g, ib, si).wait()
        out_copy(g - 2, sb, so).wait()
        expand(ib, sb)
        out_copy(g, sb, so).start()
    for b in range(2):
        ib, sb, si, so = bufs[b]
        out_copy(n - 2 + b, sb, so).wait()


def kernel(y, w1):
    B = y.shape[0] * y.shape[1]
    y_flat = y.reshape(B)

    mesh = plsc.VectorSubcoreMesh(core_axis_name="c", subcore_axis_name="s")
    run = functools.partial(
        pl.kernel,
        mesh=mesh,
        out_type=jax.ShapeDtypeStruct((B, _D), jnp.float32),
        scratch_types=[
            pltpu.VMEM((10, _D), jnp.float32),
            pltpu.VMEM((_C,), jnp.int32),
            pltpu.VMEM((_C,), jnp.int32),
            pltpu.VMEM((_C, _D), jnp.float32),
            pltpu.VMEM((_C, _D), jnp.float32),
            pltpu.SemaphoreType.DMA,
            pltpu.SemaphoreType.DMA,
            pltpu.SemaphoreType.DMA,
            pltpu.SemaphoreType.DMA,
        ],
    )(_sc_embed)
    out = run(w1, y_flat)
    return out.reshape(y.shape[0], y.shape[1], _D)


# speedup vs baseline: 25.6935x; 1.4256x over previous
import functools

import jax
import jax.numpy as jnp
from jax import lax
from jax.experimental import pallas as pl
from jax.experimental.pallas import tpu as pltpu
from jax.experimental.pallas import tpu_sc as plsc

_INFO = plsc.get_sparse_core_info()
_NC = _INFO.num_cores
_NS = _INFO.num_subcores
_NW = _NC * _NS
_L = _INFO.num_lanes

_D = 128
_C = 400


def _sc_embed(table_hbm, idx_hbm, out_hbm,
              table_v, ibuf0, ibuf1, sm0, sm1, st0, st1, si0, si1, so0, so1):
    wid = lax.axis_index("s") * _NC + lax.axis_index("c")
    B = idx_hbm.shape[0]
    per_w = B // _NW
    n = per_w // _C
    base = wid * per_w

    pltpu.sync_copy(table_hbm, table_v)

    bufs = ((ibuf0, sm0, st0, si0, so0), (ibuf1, sm1, st1, si1, so1))

    def idx_copy(g, ib, sem):
        return pltpu.make_async_copy(
            idx_hbm.at[pl.ds(base + g * _C, _C)], ib, sem)

    def out_copy(g, sb, sem):
        return pltpu.make_async_copy(
            sb, out_hbm.at[pl.ds(base + g * _C, _C)], sem)

    def stage_idx(ib, sm):
        @plsc.parallel_loop(0, _C // _L)
        def grp(g):
            r = g * _L
            tv = ib[pl.ds(r, _L)]
            for u in range(_L):
                sm[r + u] = tv[u]

    def expand(sm, sb):
        @plsc.parallel_loop(0, _C, unroll=4)
        def row(r):
            t = sm[r]
            vals = [table_v[t, pl.ds(k * _L, _L)] for k in range(_D // _L)]
            for k in range(_D // _L):
                sb[r, pl.ds(k * _L, _L)] = vals[k]

    idx_copy(0, ibuf0, si0).start()
    idx_copy(1, ibuf1, si1).start()

    for b in range(2):
        ib, sm, sb, si, so = bufs[b]
        idx_copy(b, ib, si).wait()
        stage_idx(ib, sm)
        idx_copy(b + 2, ib, si).start()
        expand(sm, sb)
        out_copy(b, sb, so).start()

    def pair(p, _):
        for b in range(2):
            g = 2 + 2 * p + b
            ib, sm, sb, si, so = bufs[b]
            idx_copy(g, ib, si).wait()
            stage_idx(ib, sm)
            idx_copy(g + 2, ib, si).start()
            out_copy(g - 2, sb, so).wait()
            expand(sm, sb)
            out_copy(g, sb, so).start()
        return ()
    lax.fori_loop(0, (n - 4) // 2, pair, ())

    for b in range(2):
        g = n - 2 + b
        ib, sm, sb, si, so = bufs[b]
        idx_copy(g, ib, si).wait()
        stage_idx(ib, sm)
        out_copy(g - 2, sb, so).wait()
        expand(sm, sb)
        out_copy(g, sb, so).start()
    for b in range(2):
        ib, sm, sb, si, so = bufs[b]
        out_copy(n - 2 + b, sb, so).wait()


def kernel(y, w1):
    B = y.shape[0] * y.shape[1]
    y_flat = y.reshape(B)

    mesh = plsc.VectorSubcoreMesh(core_axis_name="c", subcore_axis_name="s")
    run = functools.partial(
        pl.kernel,
        mesh=mesh,
        out_type=jax.ShapeDtypeStruct((B, _D), jnp.float32),
        scratch_types=[
            pltpu.VMEM((10, _D), jnp.float32),
            pltpu.VMEM((_C,), jnp.int32),
            pltpu.VMEM((_C,), jnp.int32),
            pltpu.SMEM((_C,), jnp.int32),
            pltpu.SMEM((_C,), jnp.int32),
            pltpu.VMEM((_C, _D), jnp.float32),
            pltpu.VMEM((_C, _D), jnp.float32),
            pltpu.SemaphoreType.DMA,
            pltpu.SemaphoreType.DMA,
            pltpu.SemaphoreType.DMA,
            pltpu.SemaphoreType.DMA,
        ],
    )(_sc_embed)
    out = run(w1, y_flat)
    return out.reshape(y.shape[0], y.shape[1], _D)


# speedup vs baseline: 25.7755x; 1.0032x over previous
import functools

import jax
import jax.numpy as jnp
from jax import lax
from jax.experimental import pallas as pl
from jax.experimental.pallas import tpu as pltpu
from jax.experimental.pallas import tpu_sc as plsc

_INFO = plsc.get_sparse_core_info()
_NC = _INFO.num_cores
_NS = _INFO.num_subcores
_NW = _NC * _NS
_L = _INFO.num_lanes

_D = 128
_C = 400
_CB = 4


def _sc_embed(table_hbm, idx_hbm, out_hbm,
              table_v, ib0, ib1, sm0, sm1, st0, st1, si0, si1, so0, so1):
    wid = lax.axis_index("s") * _NC + lax.axis_index("c")
    n_rows = idx_hbm.shape[0]
    per_w = n_rows // _NW
    nq = per_w // _CB
    base = wid * per_w

    pltpu.sync_copy(table_hbm, table_v)

    ibufs = (ib0, ib1)
    isems = (si0, si1)
    sbufs = ((sm0, st0, so0), (sm1, st1, so1))

    def idx_copy(j, ib, sem):
        return pltpu.make_async_copy(
            idx_hbm.at[pl.ds(base + j * _CB, _CB)], ib, sem)

    def out_copy(g, sb, sem):
        return pltpu.make_async_copy(
            sb, out_hbm.at[pl.ds((base + g) * _C, _C)], sem)

    def stage_idx(ib, s, sm):
        @plsc.parallel_loop(0, _C // _L)
        def grp(g):
            r = g * _L
            tv = ib[s, pl.ds(r, _L)]
            for u in range(_L):
                sm[r + u] = tv[u]

    def expand(sm, sb):
        @plsc.parallel_loop(0, _C, unroll=4)
        def row(r):
            t = sm[r]
            vals = [table_v[t, pl.ds(k * _L, _L)] for k in range(_D // _L)]
            for k in range(_D // _L):
                sb[r, pl.ds(k * _L, _L)] = vals[k]

    def chunk(g, s, jb, first):
        sm, sb, so = sbufs[s % 2]
        stage_idx(ibufs[jb], s, sm)
        if not first:
            out_copy(g - 2, sb, so).wait()
        expand(sm, sb)
        out_copy(g, sb, so).start()

    idx_copy(0, ib0, si0).start()
    idx_copy(1, ib1, si1).start()

    for q in range(2):
        jb = q % 2
        idx_copy(q, ibufs[jb], isems[jb]).wait()
        for s in range(_CB):
            g = q * _CB + s
            chunk(g, s, jb, first=(g < 2))
        idx_copy(q + 2, ibufs[jb], isems[jb]).start()

    def qpair(t, _):
        for a in range(2):
            q = 2 + 2 * t + a
            idx_copy(q, ibufs[a], isems[a]).wait()
            for s in range(_CB):
                chunk(q * _CB + s, s, a, first=False)
            idx_copy(q + 2, ibufs[a], isems[a]).start()
        return ()
    lax.fori_loop(0, (nq - 4) // 2, qpair, ())

    for q in (nq - 2, nq - 1):
        jb = q % 2
        idx_copy(q, ibufs[jb], isems[jb]).wait()
        for s in range(_CB):
            chunk(q * _CB + s, s, jb, first=False)
    for b in range(2):
        sm, sb, so = sbufs[b]
        out_copy(per_w - 2 + b, sb, so).wait()


def kernel(y, w1):
    B = y.shape[0] * y.shape[1]
    y2 = y.reshape(B // _C, _C)

    mesh = plsc.VectorSubcoreMesh(core_axis_name="c", subcore_axis_name="s")
    run = functools.partial(
        pl.kernel,
        mesh=mesh,
        out_type=jax.ShapeDtypeStruct((B, _D), jnp.float32),
        scratch_types=[
            pltpu.VMEM((10, _D), jnp.float32),
            pltpu.VMEM((_CB, _C), jnp.int32),
            pltpu.VMEM((_CB, _C), jnp.int32),
            pltpu.SMEM((_C,), jnp.int32),
            pltpu.SMEM((_C,), jnp.int32),
            pltpu.VMEM((_C, _D), jnp.float32),
            pltpu.VMEM((_C, _D), jnp.float32),
            pltpu.SemaphoreType.DMA,
            pltpu.SemaphoreType.DMA,
            pltpu.SemaphoreType.DMA,
            pltpu.SemaphoreType.DMA,
        ],
    )(_sc_embed)
    out = run(w1, y2)
    return out.reshape(y.shape[0], y.shape[1], _D)
